# baseline (speedup 1.0000x reference)
import functools
import jax
import jax.numpy as jnp
from jax import lax
from jax.experimental import pallas as pl
from jax.experimental.pallas import tpu as pltpu
from jax.experimental.pallas import tpu_sc as plsc

D_MODEL_K = 1024
N_EXPERTS_K = 16
TOP_K_K = 8
TOKENS_K = 2048

_info = plsc.get_sparse_core_info()
_NC, _NS, _L = _info.num_cores, _info.num_subcores, _info.num_lanes
_NW = _NC * _NS
_TOK_PER_W = TOKENS_K // _NW
_CHUNK = _TOK_PER_W * N_EXPERTS_K


def _logits_body(h_ref, gw_ref, logits_ref):
    logits_ref[...] = jax.lax.dot_general(
        h_ref[...], gw_ref[...], (((1,), (1,)), ((), ())),
        preferred_element_type=jnp.float32)


_router_mesh = plsc.VectorSubcoreMesh(core_axis_name="c", subcore_axis_name="s")


@functools.partial(
    pl.kernel,
    mesh=_router_mesh,
    compiler_params=pltpu.CompilerParams(needs_layout_passes=False),
    out_type=jax.ShapeDtypeStruct((TOKENS_K * N_EXPERTS_K,), jnp.float32),
    scratch_types=[
        pltpu.VMEM((_CHUNK,), jnp.float32),
        pltpu.VMEM((_CHUNK,), jnp.float32),
        pltpu.VMEM((_L,), jnp.float32),
    ],
)
def _sc_router(logits_hbm, comb_hbm, lg_v, cb_v, tmp_v):
    wid = lax.axis_index("s") * _NC + lax.axis_index("c")
    base = wid * _CHUNK
    pltpu.sync_copy(logits_hbm.at[pl.ds(base, _CHUNK)], lg_v)
    lane = lax.iota(jnp.int32, _L)
    ones = jnp.ones((_L,), jnp.float32)
    keep = lane < TOP_K_K

    def body(i, carry):
        l = lg_v[pl.ds(i * N_EXPERTS_K, N_EXPERTS_K)]
        _sorted, order = plsc.sort_key_val(l, lane, descending=True)
        tmp_v[...] = jnp.zeros((_L,), jnp.float32)
        plsc.store_scatter(tmp_v, [order], ones, mask=keep)
        m = tmp_v[...]
        mx = jnp.max(l)
        ex = jnp.exp(l - mx)
        w = ex / jnp.sum(ex)
        cb_v[pl.ds(i * N_EXPERTS_K, N_EXPERTS_K)] = w * m
        return carry

    lax.fori_loop(0, _TOK_PER_W, body, None)
    pltpu.sync_copy(cb_v, comb_hbm.at[pl.ds(base, _CHUNK)])


def _moe_body(h_ref, comb_ref, ew_ref, out_ref):
    e = pl.program_id(0)
    col = jax.lax.broadcasted_iota(
        jnp.int32, (h_ref.shape[0], N_EXPERTS_K), 1)
    y = jax.lax.dot_general(
        h_ref[...], ew_ref[0], (((1,), (1,)), ((), ())),
        preferred_element_type=jnp.float32)
    c = jnp.sum(jnp.where(col == e, comb_ref[...], 0.0), axis=1,
                keepdims=True)
    contrib = c * y

    @pl.when(e == 0)
    def _init():
        out_ref[...] = contrib

    @pl.when(e > 0)
    def _acc():
        out_ref[...] += contrib


@jax.jit
def kernel(hidden_states, gate_w, expert_w):
    b, t, d = hidden_states.shape
    tok = b * t
    h_flat = hidden_states.reshape(tok, d)
    n_exp = expert_w.shape[0]

    logits = pl.pallas_call(
        _logits_body,
        out_shape=jax.ShapeDtypeStruct((tok, n_exp), jnp.float32),
    )(h_flat, gate_w)

    comb = _sc_router(logits.reshape(-1)).reshape(tok, n_exp)

    out = pl.pallas_call(
        _moe_body,
        grid=(n_exp,),
        in_specs=[
            pl.BlockSpec((tok, d), lambda e: (0, 0)),
            pl.BlockSpec((tok, n_exp), lambda e: (0, 0)),
            pl.BlockSpec((1, d, d), lambda e: (e, 0, 0)),
        ],
        out_specs=pl.BlockSpec((tok, d), lambda e: (0, 0)),
        out_shape=jax.ShapeDtypeStruct((tok, d), jnp.float32),
    )(h_flat, comb, expert_w)
    return out.reshape(b, t, d), logits


# speedup vs baseline: 1.0087x; 1.0087x over previous
import functools
import jax
import jax.numpy as jnp
from jax.experimental import pallas as pl
from jax.experimental.pallas import tpu as pltpu

D_MODEL_K = 1024
N_EXPERTS_K = 16
TOP_K_K = 8


def _moe_body(h_ref, gw_ref, ew_ref, out_ref, logits_ref, comb_ref):
    e = pl.program_id(0)

    y = jax.lax.dot_general(
        h_ref[...], ew_ref[0], (((1,), (1,)), ((), ())),
        preferred_element_type=jnp.float32)

    @pl.when(e == 0)
    def _router():
        h = h_ref[...]
        logits = jax.lax.dot_general(
            h, gw_ref[...], (((1,), (1,)), ((), ())),
            preferred_element_type=jnp.float32)
        logits_ref[...] = logits
        m = jnp.max(logits, axis=1, keepdims=True)
        ex = jnp.exp(logits - m)
        w = ex / jnp.sum(ex, axis=1, keepdims=True)
        col = jax.lax.broadcasted_iota(
            jnp.int32, (h_ref.shape[0], N_EXPERTS_K), 1)
        rank = jnp.zeros(logits.shape, jnp.float32)
        for j in range(N_EXPERTS_K):
            lj = logits[:, j:j + 1]
            sel = (lj > logits) | ((lj == logits) & (j < col))
            rank += sel.astype(jnp.float32)
        comb_ref[...] = jnp.where(rank < TOP_K_K, w, 0.0)

    onehot = (jax.lax.broadcasted_iota(jnp.int32, (N_EXPERTS_K, 1), 0)
              == e).astype(jnp.float32)
    c = jax.lax.dot_general(
        comb_ref[...], onehot, (((1,), (0,)), ((), ())),
        preferred_element_type=jnp.float32)
    contrib = c * y

    @pl.when(e == 0)
    def _init():
        out_ref[...] = contrib

    @pl.when(e > 0)
    def _acc():
        out_ref[...] += contrib


@jax.jit
def kernel(hidden_states, gate_w, expert_w):
    b, t, d = hidden_states.shape
    tok = b * t
    h_flat = hidden_states.reshape(tok, d)
    n_exp = expert_w.shape[0]

    out, logits = pl.pallas_call(
        _moe_body,
        grid=(n_exp,),
        in_specs=[
            pl.BlockSpec((tok, d), lambda e: (0, 0)),
            pl.BlockSpec((n_exp, d), lambda e: (0, 0)),
            pl.BlockSpec((1, d, d), lambda e: (e, 0, 0)),
        ],
        out_specs=[
            pl.BlockSpec((tok, d), lambda e: (0, 0)),
            pl.BlockSpec((tok, n_exp), lambda e: (0, 0)),
        ],
        out_shape=[
            jax.ShapeDtypeStruct((tok, d), jnp.float32),
            jax.ShapeDtypeStruct((tok, n_exp), jnp.float32),
        ],
        scratch_shapes=[pltpu.VMEM((tok, n_exp), jnp.float32)],
    )(h_flat, gate_w, expert_w)
    return out.reshape(b, t, d), logits


# speedup vs baseline: 1.0222x; 1.0135x over previous
import functools
import jax
import jax.numpy as jnp
from jax.experimental import pallas as pl
from jax.experimental.pallas import tpu as pltpu

D_MODEL_K = 1024
N_EXPERTS_K = 16
TOP_K_K = 8


def _moe_body(h_ref, gw_ref, ew_ref, out_ref, logits_ref, comb_ref):
    e = pl.program_id(0)

    y = jax.lax.dot_general(
        h_ref[...], ew_ref[0], (((1,), (1,)), ((), ())),
        preferred_element_type=jnp.float32)

    @pl.when(e == 0)
    def _router():
        h = h_ref[...]
        logits = jax.lax.dot_general(
            h, gw_ref[...], (((1,), (1,)), ((), ())),
            preferred_element_type=jnp.float32)
        logits_ref[...] = logits
        m = jnp.max(logits, axis=1, keepdims=True)
        ex = jnp.exp(logits - m)
        w = ex / jnp.sum(ex, axis=1, keepdims=True)
        col = jax.lax.broadcasted_iota(
            jnp.int32, (h_ref.shape[0], N_EXPERTS_K), 1)
        rank = jnp.zeros(logits.shape, jnp.float32)
        for j in range(N_EXPERTS_K):
            lj = logits[:, j:j + 1]
            sel = (lj > logits) | ((lj == logits) & (j < col))
            rank += sel.astype(jnp.float32)
        comb_ref[...] = jnp.where(rank < TOP_K_K, w, 0.0)

    col2 = jax.lax.broadcasted_iota(
        jnp.int32, (h_ref.shape[0], N_EXPERTS_K), 1)
    c = jnp.sum(jnp.where(col2 == e, comb_ref[...], 0.0), axis=1,
                keepdims=True)
    contrib = c * y

    @pl.when(e == 0)
    def _init():
        out_ref[...] = contrib

    @pl.when(e > 0)
    def _acc():
        out_ref[...] += contrib


@jax.jit
def kernel(hidden_states, gate_w, expert_w):
    b, t, d = hidden_states.shape
    tok = b * t
    h_flat = hidden_states.reshape(tok, d)
    n_exp = expert_w.shape[0]

    out, logits = pl.pallas_call(
        _moe_body,
        grid=(n_exp,),
        in_specs=[
            pl.BlockSpec((tok, d), lambda e: (0, 0)),
            pl.BlockSpec((n_exp, d), lambda e: (0, 0)),
            pl.BlockSpec((1, d, d), lambda e: (e, 0, 0)),
        ],
        out_specs=[
            pl.BlockSpec((tok, d), lambda e: (0, 0)),
            pl.BlockSpec((tok, n_exp), lambda e: (0, 0)),
        ],
        out_shape=[
            jax.ShapeDtypeStruct((tok, d), jnp.float32),
            jax.ShapeDtypeStruct((tok, n_exp), jnp.float32),
        ],
        scratch_shapes=[pltpu.VMEM((tok, n_exp), jnp.float32)],
    )(h_flat, gate_w, expert_w)
    return out.reshape(b, t, d), logits


# speedup vs baseline: 1.1397x; 1.1149x over previous
import functools
import jax
import jax.numpy as jnp
from jax.experimental import pallas as pl
from jax.experimental.pallas import tpu as pltpu

D_MODEL_K = 1024
N_EXPERTS_K = 16
TOP_K_K = 8


def _moe_body(h_ref, gw_ref, ew_ref, out_ref, logits_ref, comb_ref):
    e = pl.program_id(0)
    col = jax.lax.broadcasted_iota(jnp.int32, (h_ref.shape[0], N_EXPERTS_K), 1)

    @pl.when(e == 0)
    def _router():
        h = h_ref[...]
        logits = jax.lax.dot_general(
            h, gw_ref[...], (((1,), (1,)), ((), ())),
            preferred_element_type=jnp.float32)
        logits_ref[...] = logits
        m = jnp.max(logits, axis=1, keepdims=True)
        ex = jnp.exp(logits - m)
        w = ex / jnp.sum(ex, axis=1, keepdims=True)
        rank = jnp.zeros(logits.shape, jnp.int32)
        for j in range(N_EXPERTS_K):
            lj = logits[:, j:j + 1]
            rank += (lj > logits).astype(jnp.int32)
            rank += ((lj == logits) & (j < col)).astype(jnp.int32)
        comb_ref[...] = jnp.where(rank < TOP_K_K, w, 0.0)

    y = jax.lax.dot_general(
        h_ref[...], ew_ref[0], (((1,), (1,)), ((), ())),
        preferred_element_type=jnp.float32)
    c = jnp.sum(jnp.where(col == e, comb_ref[...], 0.0), axis=1, keepdims=True)
    contrib = c * y

    @pl.when(e == 0)
    def _init():
        out_ref[...] = contrib

    @pl.when(e > 0)
    def _acc():
        out_ref[...] += contrib


@jax.jit
def kernel(hidden_states, gate_w, expert_w):
    b, t, d = hidden_states.shape
    h_flat = hidden_states.reshape(t * b, d)
    n_exp = expert_w.shape[0]

    out, logits = pl.pallas_call(
        _moe_body,
        grid=(n_exp,),
        in_specs=[
            pl.BlockSpec((t * b, d), lambda e: (0, 0)),
            pl.BlockSpec((n_exp, d), lambda e: (0, 0)),
            pl.BlockSpec((1, d, d), lambda e: (e, 0, 0)),
        ],
        out_specs=[
            pl.BlockSpec((t * b, d), lambda e: (0, 0)),
            pl.BlockSpec((t * b, n_exp), lambda e: (0, 0)),
        ],
        out_shape=[
            jax.ShapeDtypeStruct((t * b, d), jnp.float32),
            jax.ShapeDtypeStruct((t * b, n_exp), jnp.float32),
        ],
        scratch_shapes=[pltpu.VMEM((t * b, n_exp), jnp.float32)],
    )(h_flat, gate_w, expert_w)
    return out.reshape(b, t, d), logits
